# baseline (speedup 1.0000x reference)
import jax
import jax.numpy as jnp
from jax import lax
from jax.experimental import pallas as pl
from jax.experimental.pallas import tpu as pltpu
from jax.experimental.pallas import tpu_sc as plsc

NC = 2
NS = 16
NW = NC * NS
D = 1024
PER_W = 1024

CA = 64
CB = 56

_SIZES = [CA, CB] * 8 + [CA]
_OFFS = [sum(_SIZES[:j]) for j in range(len(_SIZES))]
NCHUNK = len(_SIZES)


def _gather_body(x_hbm, pe_hbm, out_hbm, idx_v, buf_a, buf_b,
                 ga, gb, sa, sb):
    bufs = (buf_a, buf_b)
    gsems = (ga, gb)
    ssems = (sa, sb)

    cid = lax.axis_index("c")
    sid = lax.axis_index("s")
    wid = sid * NC + cid
    base = wid * PER_W

    pltpu.sync_copy(x_hbm.at[wid], idx_v)

    def gather(j):
        b = j % 2
        pltpu.async_copy(pe_hbm.at[idx_v.at[pl.ds(_OFFS[j], _SIZES[j])]],
                         bufs[b], gsems[b])

    def gather_wait(j):
        b = j % 2
        pltpu.make_async_copy(pe_hbm.at[pl.ds(0, _SIZES[j])], bufs[b],
                              gsems[b]).wait()

    def store(j):
        b = j % 2
        off = pl.multiple_of(base + _OFFS[j], 8)
        pltpu.async_copy(bufs[b], out_hbm.at[pl.ds(off, _SIZES[j])],
                         ssems[b])

    def store_wait(j):
        b = j % 2
        pltpu.make_async_copy(bufs[b], out_hbm.at[pl.ds(0, _SIZES[j])],
                              ssems[b]).wait()

    gather(0)
    for j in range(NCHUNK):
        gather_wait(j)
        store(j)
        if j + 1 < NCHUNK:
            if j >= 1:
                store_wait(j - 1)
            gather(j + 1)
    store_wait(NCHUNK - 2)
    store_wait(NCHUNK - 1)


@jax.jit
def _sc_gather(x_resh, pe):
    mesh = plsc.VectorSubcoreMesh(core_axis_name="c", subcore_axis_name="s")
    run = pl.kernel(
        _gather_body,
        out_type=jax.ShapeDtypeStruct((NW * PER_W, D), jnp.float32),
        mesh=mesh,
        scratch_types=[
            pltpu.VMEM((PER_W,), jnp.int32),
            pltpu.VMEM((CA, D), jnp.float32),
            pltpu.VMEM((CB, D), jnp.float32),
            pltpu.SemaphoreType.DMA,
            pltpu.SemaphoreType.DMA,
            pltpu.SemaphoreType.DMA,
            pltpu.SemaphoreType.DMA,
        ],
    )
    return run(x_resh, pe)


def kernel(x, pe):
    B, S = x.shape
    x_resh = x.reshape(NW, PER_W)
    out = _sc_gather(x_resh, pe)
    return out.reshape(B, S, D)


# speedup vs baseline: 1.0164x; 1.0164x over previous
import jax
import jax.numpy as jnp
from jax import lax
from jax.experimental import pallas as pl
from jax.experimental.pallas import tpu as pltpu
from jax.experimental.pallas import tpu_sc as plsc

NC = 2
NS = 16
NW = NC * NS
D = 1024
CHUNK = 32
NBUF = 2


def _gather_body(x_hbm, pe_hbm, out_hbm, idx_v, *rest):
    nch = idx_v.shape[0]
    bufs = rest[:NBUF]
    gsems = rest[NBUF:2 * NBUF]
    ssems = rest[2 * NBUF:3 * NBUF]

    cid = lax.axis_index("c")
    sid = lax.axis_index("s")
    wid = sid * NC + cid

    pltpu.sync_copy(x_hbm.at[wid], idx_v)

    pltpu.async_copy(pe_hbm.at[idx_v.at[0]], bufs[0], gsems[0])

    def outer(i, carry):
        for b in range(NBUF):
            g = i * NBUF + b
            nb = (b + 1) % NBUF
            pltpu.make_async_copy(pe_hbm.at[pl.ds(0, CHUNK)], bufs[b],
                                  gsems[b]).wait()
            pltpu.async_copy(bufs[b], out_hbm.at[wid, g], ssems[b])
            if b < NBUF - 1:
                @pl.when(i >= 1)
                def _():
                    pltpu.make_async_copy(bufs[nb], out_hbm.at[wid, 0],
                                          ssems[nb]).wait()

                pltpu.async_copy(pe_hbm.at[idx_v.at[g + 1]], bufs[nb],
                                 gsems[nb])
            else:
                @pl.when(g + 1 < nch)
                def _():
                    pltpu.make_async_copy(bufs[nb], out_hbm.at[wid, 0],
                                          ssems[nb]).wait()
                    pltpu.async_copy(pe_hbm.at[idx_v.at[g + 1]], bufs[nb],
                                     gsems[nb])

        return carry

    lax.fori_loop(0, nch // NBUF, outer, 0)

    for b in range(NBUF):
        pltpu.make_async_copy(bufs[b], out_hbm.at[wid, 0], ssems[b]).wait()


@jax.jit
def _sc_gather(x_resh, pe):
    nch = x_resh.shape[1]
    mesh = plsc.VectorSubcoreMesh(core_axis_name="c", subcore_axis_name="s")
    scratch = (
        [pltpu.VMEM((nch, CHUNK), jnp.int32)]
        + [pltpu.VMEM((CHUNK, D), jnp.float32) for _ in range(NBUF)]
        + [pltpu.SemaphoreType.DMA for _ in range(2 * NBUF)]
    )
    run = pl.kernel(
        _gather_body,
        out_type=jax.ShapeDtypeStruct((NW, nch, CHUNK, D), jnp.float32),
        mesh=mesh,
        scratch_types=scratch,
    )
    return run(x_resh, pe)


def kernel(x, pe):
    B, S = x.shape
    total = B * S
    per_w = total // NW
    nch = per_w // CHUNK
    x_resh = x.reshape(NW, nch, CHUNK)
    out = _sc_gather(x_resh, pe)
    return out.reshape(B, S, D)
